# baseline (speedup 1.0000x reference)
import functools

import jax
import jax.numpy as jnp
from jax import lax
from jax.experimental import pallas as pl
from jax.experimental.pallas import tpu as pltpu
from jax.experimental.pallas import tpu_sc as plsc

BATCH = 4096
SEQ = 200
VOCAB = 100000

NC = 2
NS = 16
L = 16
NW = NC * NS
COLS_W = BATCH // NW
RCHUNK = 40
NCHUNK = SEQ // RCHUNK
VPR = COLS_W // L


@jax.jit
def _sc_gather(tokens_t, table):
    mesh = plsc.VectorSubcoreMesh(core_axis_name="c", subcore_axis_name="s")

    @functools.partial(
        pl.kernel,
        out_type=jax.ShapeDtypeStruct((SEQ, BATCH), jnp.float32),
        mesh=mesh,
        compiler_params=pltpu.CompilerParams(needs_layout_passes=False),
        scratch_types=[
            pltpu.VMEM((VOCAB,), jnp.float32),
            pltpu.VMEM_SHARED((VOCAB,), jnp.float32),
            pltpu.VMEM((RCHUNK, COLS_W), jnp.int32),
            pltpu.VMEM((RCHUNK, COLS_W), jnp.int32),
            pltpu.VMEM((RCHUNK, COLS_W), jnp.float32),
            pltpu.VMEM((RCHUNK, COLS_W), jnp.float32),
            pltpu.SemaphoreType.DMA,
            pltpu.SemaphoreType.DMA((2,)),
            pltpu.SemaphoreType.DMA((2,)),
        ],
    )
    def k(tokens_hbm, table_hbm, out_hbm, table_v, table_sp, tok0, tok1,
          out0, out1, tsem, tok_sems, out_sems):
        toks = [tok0, tok1]
        outs = [out0, out1]
        wid = lax.axis_index("s") * NC + lax.axis_index("c")
        col0 = wid * COLS_W

        def tok_window(c):
            return tokens_hbm.at[pl.ds(c * RCHUNK, RCHUNK),
                                 pl.ds(col0, COLS_W)]

        def out_window(c):
            return out_hbm.at[pl.ds(c * RCHUNK, RCHUNK), pl.ds(col0, COLS_W)]

        tok_dmas = [None] * NCHUNK
        out_dmas = [None] * NCHUNK
        tok_dmas[0] = pltpu.async_copy(tok_window(0), tok0, tok_sems.at[0])
        sid = lax.axis_index("s")

        @pl.when(sid == 0)
        def _():
            pltpu.sync_copy(table_hbm, table_sp)

        if NCHUNK > 1:
            tok_dmas[1] = pltpu.async_copy(tok_window(1), tok1, tok_sems.at[1])
        plsc.subcore_barrier()
        pltpu.sync_copy(table_sp, table_v)

        for c in range(NCHUNK):
            b = c & 1
            tok_dmas[c].wait()
            if c >= 2:
                out_dmas[c - 2].wait()

            @plsc.parallel_loop(0, RCHUNK, unroll=2)
            def row_loop(r, tok_v=toks[b], out_v=outs[b]):
                for v in range(VPR):
                    idx = tok_v[r, pl.ds(v * L, L)]
                    out_v[r, pl.ds(v * L, L)] = plsc.load_gather(
                        table_v, [idx])

            out_dmas[c] = pltpu.async_copy(outs[b], out_window(c),
                                           out_sems.at[b])
            if c + 2 < NCHUNK:
                tok_dmas[c + 2] = pltpu.async_copy(tok_window(c + 2), toks[b],
                                                   tok_sems.at[b])

        for c in range(max(0, NCHUNK - 2), NCHUNK):
            out_dmas[c].wait()

    return k(tokens_t, table)


def kernel(tokens, table):
    return _sc_gather(tokens.T, table).T
